# initial kernel scaffold (unmeasured)
import jax
import jax.numpy as jnp
from jax import lax
from jax.experimental import pallas as pl
from jax.experimental.pallas import tpu as pltpu

N_DEV = 8
B, Sq, Hq, Dh = 2, 512, 8, 64
S_KV = 512
D_MODEL = 768
D_QK = Hq * Dh
BLK = 64
NCHUNK = 1
S_C = S_KV // NCHUNK


def kernel(x, Wq, K_ext, V_ext, Wo):
    def body(x_ref, wq_ref, k_ref, v_ref, wo_ref, out_ref,
             kv_ref, ctx_ref, send_sems, recv_sems):
        my = lax.axis_index("i")

        def mk(c, dev):
            return pltpu.make_async_remote_copy(
                src_ref=kv_ref.at[c],
                dst_ref=kv_ref.at[c],
                send_sem=send_sems.at[c],
                recv_sem=recv_sems.at[c],
                device_id=(dev,),
                device_id_type=pl.DeviceIdType.MESH,
            )

        xm = x_ref[...].reshape(B * Sq, D_MODEL).astype(jnp.bfloat16)
        wq = wq_ref[...].astype(jnp.bfloat16)
        q = lax.dot_general(xm, wq, (((1,), (0,)), ((), ())),
                            preferred_element_type=jnp.float32) * 0.125
        q = q.astype(jnp.bfloat16)

        @pl.when(my == 0)
        def _():
            for c in range(NCHUNK):
                sl = slice(c * S_C, (c + 1) * S_C)
                for b in range(B):
                    for h in range(Hq):
                        kv_ref[c, 0, b, h] = k_ref[b, sl, h, :].astype(jnp.bfloat16)
                        kv_ref[c, 1, b, h] = v_ref[b, sl, h, :].astype(jnp.bfloat16)
            for c in range(NCHUNK):
                mk(c, my + 1).start()
            for c in range(NCHUNK):
                mk(c, my + 1).wait_send()

        @pl.when((my > 0) & (my < N_DEV - 1))
        def _():
            for c in range(NCHUNK):
                mk(c, my - 1).wait_recv()
                mk(c, my + 1).start()
            for c in range(NCHUNK):
                mk(c, my + 1).wait_send()

        @pl.when(my == N_DEV - 1)
        def _():
            for c in range(NCHUNK):
                mk(c, my - 1).wait_recv()

        kv = kv_ref[...]
        rows = lax.broadcasted_iota(jnp.int32, (Sq, S_KV), 0)
        cols = lax.broadcasted_iota(jnp.int32, (Sq, S_KV), 1)
        mask = (cols // BLK) <= (rows // BLK)
        for b in range(B):
            for h in range(Hq):
                if NCHUNK == 1:
                    kbh = kv[0, 0, b, h]
                    vbh = kv[0, 1, b, h]
                else:
                    kbh = jnp.concatenate(
                        [kv[c, 0, b, h] for c in range(NCHUNK)], axis=0)
                    vbh = jnp.concatenate(
                        [kv[c, 1, b, h] for c in range(NCHUNK)], axis=0)
                qbh = q[b * Sq:(b + 1) * Sq, h * Dh:(h + 1) * Dh]
                s = lax.dot_general(qbh, kbh, (((1,), (1,)), ((), ())),
                                    preferred_element_type=jnp.float32)
                s = jnp.where(mask, s, -1e9)
                m = jnp.max(s, axis=-1, keepdims=True)
                w = jnp.exp(s - m)
                p = (w / jnp.sum(w, axis=-1, keepdims=True)).astype(jnp.bfloat16)
                ctx = lax.dot_general(p, vbh, (((1,), (0,)), ((), ())),
                                      preferred_element_type=jnp.float32)
                ctx_ref[b * Sq:(b + 1) * Sq, h * Dh:(h + 1) * Dh] = (
                    ctx.astype(jnp.bfloat16))

        out = lax.dot_general(ctx_ref[...], wo_ref[...].astype(jnp.bfloat16),
                              (((1,), (0,)), ((), ())),
                              preferred_element_type=jnp.float32)
        out_ref[...] = out.reshape(B, Sq, D_MODEL)

    return pl.pallas_call(
        body,
        out_shape=jax.ShapeDtypeStruct((B, Sq, D_MODEL), jnp.float32),
        in_specs=[pl.BlockSpec(memory_space=pltpu.VMEM)] * 5,
        out_specs=pl.BlockSpec(memory_space=pltpu.VMEM),
        scratch_shapes=[
            pltpu.VMEM((NCHUNK, 2, B, Hq, S_C, Dh), jnp.bfloat16),
            pltpu.VMEM((B * Sq, D_QK), jnp.bfloat16),
            pltpu.SemaphoreType.DMA((NCHUNK,)),
            pltpu.SemaphoreType.DMA((NCHUNK,)),
        ],
        compiler_params=pltpu.CompilerParams(collective_id=0),
    )(x, Wq, K_ext, V_ext, Wo)


# baseline (device time: 216553 ns/iter reference)
import jax
import jax.numpy as jnp
from jax import lax
from jax.experimental import pallas as pl
from jax.experimental.pallas import tpu as pltpu

N_DEV = 8
B, Sq, Hq, Dh = 2, 512, 8, 64
S_KV = 512
D_MODEL = 768
D_QK = Hq * Dh
BLK = 64
NCHUNK = 1
S_C = S_KV // NCHUNK


def kernel(x, Wq, K_ext, V_ext, Wo):
    def body(x_ref, wq_ref, k_ref, v_ref, wo_ref, out_ref,
             kv_ref, ctx_ref, send_sems, recv_sems):
        my = lax.axis_index("i")

        def mk(c, dev):
            return pltpu.make_async_remote_copy(
                src_ref=kv_ref.at[c],
                dst_ref=kv_ref.at[c],
                send_sem=send_sems.at[c],
                recv_sem=recv_sems.at[c],
                device_id=(dev,),
                device_id_type=pl.DeviceIdType.MESH,
            )

        xm = x_ref[...].reshape(B * Sq, D_MODEL).astype(jnp.bfloat16)
        wq = wq_ref[...].astype(jnp.bfloat16)
        q = lax.dot_general(xm, wq, (((1,), (0,)), ((), ())),
                            preferred_element_type=jnp.float32) * 0.125
        q = q.astype(jnp.bfloat16)

        @pl.when(my == 0)
        def _():
            for c in range(NCHUNK):
                sl = slice(c * S_C, (c + 1) * S_C)
                for b in range(B):
                    for h in range(Hq):
                        kv_ref[c, 0, b, h] = k_ref[b, sl, h, :].astype(jnp.bfloat16)
                        kv_ref[c, 1, b, h] = v_ref[b, sl, h, :].astype(jnp.bfloat16)
            for c in range(NCHUNK):
                mk(c, my + 1).start()
            for c in range(NCHUNK):
                mk(c, my + 1).wait_send()

        @pl.when((my > 0) & (my < N_DEV - 1))
        def _():
            for c in range(NCHUNK):
                mk(c, my - 1).wait_recv()
                mk(c, my + 1).start()
            for c in range(NCHUNK):
                mk(c, my + 1).wait_send()

        @pl.when(my == N_DEV - 1)
        def _():
            for c in range(NCHUNK):
                mk(c, my - 1).wait_recv()

        kv = kv_ref[...]
        rows = lax.broadcasted_iota(jnp.int32, (Sq, S_KV), 0)
        cols = lax.broadcasted_iota(jnp.int32, (Sq, S_KV), 1)
        mask = (cols // BLK) <= (rows // BLK)
        for b in range(B):
            for h in range(Hq):
                if NCHUNK == 1:
                    kbh = kv[0, 0, b, h]
                    vbh = kv[0, 1, b, h]
                else:
                    kbh = jnp.concatenate(
                        [kv[c, 0, b, h] for c in range(NCHUNK)], axis=0)
                    vbh = jnp.concatenate(
                        [kv[c, 1, b, h] for c in range(NCHUNK)], axis=0)
                qbh = q[b * Sq:(b + 1) * Sq, h * Dh:(h + 1) * Dh]
                s = lax.dot_general(qbh, kbh, (((1,), (1,)), ((), ())),
                                    preferred_element_type=jnp.float32)
                s = jnp.where(mask, s, -1e9)
                m = jnp.max(s, axis=-1, keepdims=True)
                w = jnp.exp(s - m)
                p = (w / jnp.sum(w, axis=-1, keepdims=True)).astype(jnp.bfloat16)
                ctx = lax.dot_general(p, vbh, (((1,), (0,)), ((), ())),
                                      preferred_element_type=jnp.float32)
                ctx_ref[b * Sq:(b + 1) * Sq, h * Dh:(h + 1) * Dh] = (
                    ctx.astype(jnp.bfloat16))

        out = lax.dot_general(ctx_ref[...], wo_ref[...].astype(jnp.bfloat16),
                              (((1,), (0,)), ((), ())),
                              preferred_element_type=jnp.float32)
        out_ref[...] = out.reshape(B, Sq, D_MODEL)

    return pl.pallas_call(
        body,
        out_shape=jax.ShapeDtypeStruct((B, Sq, D_MODEL), jnp.float32),
        in_specs=[pl.BlockSpec(memory_space=pltpu.VMEM)] * 5,
        out_specs=pl.BlockSpec(memory_space=pltpu.VMEM),
        scratch_shapes=[
            pltpu.VMEM((NCHUNK, 2, B, Hq, S_C, Dh), jnp.bfloat16),
            pltpu.VMEM((B * Sq, D_QK), jnp.bfloat16),
            pltpu.SemaphoreType.DMA((NCHUNK,)),
            pltpu.SemaphoreType.DMA((NCHUNK,)),
        ],
    )(x, Wq, K_ext, V_ext, Wo)


# device time: 92648 ns/iter; 2.3374x vs baseline; 2.3374x over previous
import jax
import jax.numpy as jnp
from jax import lax
from jax.experimental import pallas as pl
from jax.experimental.pallas import tpu as pltpu

N_DEV = 8
B, Sq, Hq, Dh = 2, 512, 8, 64
S_KV = 512
D_MODEL = 768
D_QK = Hq * Dh
BLK = 64
NCHUNK = Hq


def kernel(x, Wq, K_ext, V_ext, Wo):
    def body(x_ref, wq_ref, k_ref, v_ref, wo_ref, out_ref,
             kv_ref, ctx_ref, send_sems, recv_sems):
        my = lax.axis_index("i")

        def mk(c, dev, sem_row=0):
            return pltpu.make_async_remote_copy(
                src_ref=kv_ref.at[c],
                dst_ref=kv_ref.at[c],
                send_sem=send_sems.at[sem_row, c],
                recv_sem=recv_sems.at[c],
                device_id=(dev,),
                device_id_type=pl.DeviceIdType.MESH,
            )

        xm = x_ref[...].reshape(B * Sq, D_MODEL).astype(jnp.bfloat16)
        wq = wq_ref[...].astype(jnp.bfloat16)
        q = lax.dot_general(xm, wq, (((1,), (0,)), ((), ())),
                            preferred_element_type=jnp.float32) * 0.125
        q = q.astype(jnp.bfloat16)

        @pl.when(my == 0)
        def _():
            for h in range(Hq):
                for b in range(B):
                    kv_ref[h, 0, b] = k_ref[b, :, h, :].astype(jnp.bfloat16)
                    kv_ref[h, 1, b] = v_ref[b, :, h, :].astype(jnp.bfloat16)
            for c in range(NCHUNK):
                mk(c, 1, 0).start()
                mk(c, 4, 1).start()
            for c in range(NCHUNK):
                mk(c, 1, 0).wait_send()
                mk(c, 4, 1).wait_send()

        is_fwd = ((my == 1) | (my == 2)) | ((my == 4) | (my == 5) | (my == 6))

        @pl.when(is_fwd)
        def _():
            for c in range(NCHUNK):
                mk(c, my - 1).wait_recv()
                mk(c, my + 1).start()
            for c in range(NCHUNK):
                mk(c, my + 1).wait_send()

        @pl.when((my == 3) | (my == N_DEV - 1))
        def _():
            for c in range(NCHUNK):
                mk(c, my - 1).wait_recv()

        kv = kv_ref[...]
        rows = lax.broadcasted_iota(jnp.int32, (Sq, S_KV), 0)
        cols = lax.broadcasted_iota(jnp.int32, (Sq, S_KV), 1)
        mask = (cols // BLK) <= (rows // BLK)
        for b in range(B):
            for h in range(Hq):
                kbh = kv[h, 0, b]
                vbh = kv[h, 1, b]
                qbh = q[b * Sq:(b + 1) * Sq, h * Dh:(h + 1) * Dh]
                s = lax.dot_general(qbh, kbh, (((1,), (1,)), ((), ())),
                                    preferred_element_type=jnp.float32)
                s = jnp.where(mask, s, -1e9)
                m = jnp.max(s, axis=-1, keepdims=True)
                w = jnp.exp(s - m)
                p = (w / jnp.sum(w, axis=-1, keepdims=True)).astype(jnp.bfloat16)
                ctx = lax.dot_general(p, vbh, (((1,), (0,)), ((), ())),
                                      preferred_element_type=jnp.float32)
                ctx_ref[b * Sq:(b + 1) * Sq, h * Dh:(h + 1) * Dh] = (
                    ctx.astype(jnp.bfloat16))

        out = lax.dot_general(ctx_ref[...], wo_ref[...].astype(jnp.bfloat16),
                              (((1,), (0,)), ((), ())),
                              preferred_element_type=jnp.float32)
        out_ref[...] = out.reshape(B, Sq, D_MODEL)

    return pl.pallas_call(
        body,
        out_shape=jax.ShapeDtypeStruct((B, Sq, D_MODEL), jnp.float32),
        in_specs=[pl.BlockSpec(memory_space=pltpu.VMEM)] * 5,
        out_specs=pl.BlockSpec(memory_space=pltpu.VMEM),
        scratch_shapes=[
            pltpu.VMEM((Hq, 2, B, S_KV, Dh), jnp.bfloat16),
            pltpu.VMEM((B * Sq, D_QK), jnp.bfloat16),
            pltpu.SemaphoreType.DMA((2, NCHUNK)),
            pltpu.SemaphoreType.DMA((NCHUNK,)),
        ],
    )(x, Wq, K_ext, V_ext, Wo)


# device time: 73107 ns/iter; 2.9621x vs baseline; 1.2673x over previous
import jax
import jax.numpy as jnp
from jax import lax
from jax.experimental import pallas as pl
from jax.experimental.pallas import tpu as pltpu

N_DEV = 8
B, Sq, Hq, Dh = 2, 512, 8, 64
S_KV = 512
D_MODEL = 768
D_QK = Hq * Dh
BLK = 64
NCHUNK = Hq

_DOWN = {0: (1, 3, 4), 1: (2,), 2: (), 3: (7,), 4: (5,), 5: (6,), 6: (), 7: ()}
_UP = {1: 0, 2: 1, 3: 0, 4: 0, 5: 4, 6: 5, 7: 3}
_PARTNERS = {
    d: tuple(sorted(set(_DOWN[d]) | ({_UP[d]} if d in _UP else set())))
    for d in range(N_DEV)
}


def kernel(x, Wq, K_ext, V_ext, Wo):
    def body(x_ref, wq_ref, k_ref, v_ref, wo_ref, out_ref,
             kv_ref, ctx_ref, send_sems, recv_sems):
        my = lax.axis_index("i")

        def mk(c, dev, sem_row=0):
            return pltpu.make_async_remote_copy(
                src_ref=kv_ref.at[c],
                dst_ref=kv_ref.at[c],
                send_sem=send_sems.at[sem_row, c],
                recv_sem=recv_sems.at[c],
                device_id=(dev,),
                device_id_type=pl.DeviceIdType.MESH,
            )

        barrier_sem = pltpu.get_barrier_semaphore()
        for d in range(N_DEV):
            @pl.when(my == d)
            def _(d=d):
                for p in _PARTNERS[d]:
                    pl.semaphore_signal(
                        barrier_sem, inc=1,
                        device_id=(p,), device_id_type=pl.DeviceIdType.MESH)
                pl.semaphore_wait(barrier_sem, len(_PARTNERS[d]))

        @pl.when(my == 0)
        def _():
            for h in range(Hq):
                for b in range(B):
                    kv_ref[h, 0, b] = k_ref[b, :, h, :].astype(jnp.bfloat16)
                    kv_ref[h, 1, b] = v_ref[b, :, h, :].astype(jnp.bfloat16)
            for h in range(NCHUNK):
                for row, tgt in enumerate(_DOWN[0]):
                    mk(h, tgt, row).start()

        xm = x_ref[...].reshape(B * Sq, D_MODEL).astype(jnp.bfloat16)
        wq = wq_ref[...].astype(jnp.bfloat16)
        q = lax.dot_general(xm, wq, (((1,), (0,)), ((), ())),
                            preferred_element_type=jnp.float32) * 0.125
        q = q.astype(jnp.bfloat16)

        rows = lax.broadcasted_iota(jnp.int32, (Sq, S_KV), 0)
        cols = lax.broadcasted_iota(jnp.int32, (Sq, S_KV), 1)
        mask = (cols // BLK) <= (rows // BLK)

        is_fwd = (my == 1) | (my == 3) | (my == 4) | (my == 5)

        for h in range(Hq):
            @pl.when(my != 0)
            def _(h=h):
                mk(h, 0).wait_recv()

            @pl.when(is_fwd)
            def _(h=h):
                tgt = jnp.where(my == 3, 7, my + 1)
                mk(h, tgt).start()

            for b in range(B):
                kbh = kv_ref[h, 0, b]
                vbh = kv_ref[h, 1, b]
                qbh = q[b * Sq:(b + 1) * Sq, h * Dh:(h + 1) * Dh]
                s = lax.dot_general(qbh, kbh, (((1,), (1,)), ((), ())),
                                    preferred_element_type=jnp.float32)
                s = jnp.where(mask, s, -1e9)
                w = jnp.exp(s)
                ssum = jnp.sum(w, axis=-1, keepdims=True)
                ctx = lax.dot_general(w.astype(jnp.bfloat16), vbh,
                                      (((1,), (0,)), ((), ())),
                                      preferred_element_type=jnp.float32)
                ctx = ctx * (1.0 / ssum)
                ctx_ref[b * Sq:(b + 1) * Sq, h * Dh:(h + 1) * Dh] = (
                    ctx.astype(jnp.bfloat16))

        out = lax.dot_general(ctx_ref[...], wo_ref[...].astype(jnp.bfloat16),
                              (((1,), (0,)), ((), ())),
                              preferred_element_type=jnp.float32)
        out_ref[...] = out.reshape(B, Sq, D_MODEL)

        @pl.when(my == 0)
        def _():
            for h in range(NCHUNK):
                for row, tgt in enumerate(_DOWN[0]):
                    mk(h, tgt, row).wait_send()

        @pl.when(is_fwd)
        def _():
            for h in range(NCHUNK):
                tgt = jnp.where(my == 3, 7, my + 1)
                mk(h, tgt).wait_send()

    return pl.pallas_call(
        body,
        out_shape=jax.ShapeDtypeStruct((B, Sq, D_MODEL), jnp.float32),
        in_specs=[pl.BlockSpec(memory_space=pltpu.VMEM)] * 5,
        out_specs=pl.BlockSpec(memory_space=pltpu.VMEM),
        scratch_shapes=[
            pltpu.VMEM((Hq, 2, B, S_KV, Dh), jnp.bfloat16),
            pltpu.VMEM((B * Sq, D_QK), jnp.bfloat16),
            pltpu.SemaphoreType.DMA((3, NCHUNK)),
            pltpu.SemaphoreType.DMA((NCHUNK,)),
        ],
        compiler_params=pltpu.CompilerParams(collective_id=0),
    )(x, Wq, K_ext, V_ext, Wo)
